# Initial kernel scaffold; baseline (speedup 1.0000x reference)
import jax
import jax.numpy as jnp
from jax.experimental import pallas as pl


def kernel(feat, W, b):
    raise NotImplementedError("write your pallas kernel here")


# speedup vs baseline: 4.3722x; 4.3722x over previous
import functools

import jax
import jax.numpy as jnp
from jax import lax
from jax.experimental import pallas as pl
from jax.experimental.pallas import tpu as pltpu
from jax.experimental.pallas import tpu_sc as plsc

B, C, N, K = 4, 128, 2048, 16
NKP = 1024
DUMP = B * NKP

_PREC = "default"


_NB_A = 256


def _knn_body(xt_ref, x_ref, out_ref):
    b = pl.program_id(0)
    xtb = xt_ref[0]
    xb = x_ref[0]
    inner = -2.0 * jnp.dot(xtb, xb, precision=_PREC,
                           preferred_element_type=jnp.float32)
    xx_row = jnp.sum(xb * xb, axis=0, keepdims=True)
    xx_col = jnp.sum(xtb * xtb, axis=1, keepdims=True)
    dwork = -xx_col - inner - xx_row

    iota = lax.broadcasted_iota(jnp.int32, (_NB_A, N), 1)
    neg_inf = jnp.float32(-jnp.inf)
    bigi = jnp.int32(1 << 30)
    cols = []
    for t in range(K):
        rowmax = jnp.max(dwork, axis=1, keepdims=True)
        cand = jnp.where(dwork == rowmax, iota, bigi)
        mstar = jnp.min(cand, axis=1, keepdims=True)
        cols.append(mstar)
        if t < K - 1:
            dwork = jnp.where(iota == mstar, neg_inf, dwork)
    out_ref[0] = jnp.concatenate(cols, axis=1) + b * N


_knn_call = pl.pallas_call(
    _knn_body,
    grid=(B, N // _NB_A),
    in_specs=[
        pl.BlockSpec((1, _NB_A, C), lambda b, i: (b, i, 0)),
        pl.BlockSpec((1, C, N), lambda b, i: (b, 0, 0)),
    ],
    out_specs=pl.BlockSpec((1, _NB_A, K), lambda b, i: (b, i, 0)),
    out_shape=jax.ShapeDtypeStruct((B, N, K), jnp.int32),
)


_NB_C = 512


def _score_body(nbr_ref, xt_ref, w_ref, b_ref, out_ref):
    j = pl.program_id(2)
    nbrb = nbr_ref[0, 0]
    xtb = xt_ref[0]
    gf = jnp.concatenate([nbrb - xtb, xtb], axis=1)
    sc = jnp.dot(gf, w_ref[...], precision=_PREC,
                 preferred_element_type=jnp.float32) + b_ref[0, 0]

    @pl.when(j == 0)
    def _():
        out_ref[0] = sc

    @pl.when(j > 0)
    def _():
        out_ref[0] = jnp.maximum(out_ref[0], sc)


_score_call = pl.pallas_call(
    _score_body,
    grid=(B, N // _NB_C, K),
    in_specs=[
        pl.BlockSpec((1, 1, _NB_C, C), lambda b, i, j: (j, b, i, 0)),
        pl.BlockSpec((1, _NB_C, C), lambda b, i, j: (b, i, 0)),
        pl.BlockSpec((2 * C, 1), lambda b, i, j: (0, 0)),
        pl.BlockSpec((1, 1), lambda b, i, j: (0, 0)),
    ],
    out_specs=pl.BlockSpec((1, _NB_C, 1), lambda b, i, j: (b, i, 0)),
    out_shape=jax.ShapeDtypeStruct((B, N, 1), jnp.float32),
)


_NB_D = 512


def _rank_body(sc_ref, sr_ref, xt_ref, dest_ref, scaled_ref):
    b = pl.program_id(0)
    i = pl.program_id(1)
    s_col = jnp.maximum(sc_ref[0], 0.0)
    s_row = jnp.maximum(sr_ref[0], 0.0)
    gt = (s_row > s_col).astype(jnp.int32)
    ncol = i * _NB_D + lax.broadcasted_iota(jnp.int32, (_NB_D, 1), 0)
    mrow = lax.broadcasted_iota(jnp.int32, (_NB_D, N), 1)
    eqlt = ((s_row == s_col) & (mrow < ncol)).astype(jnp.int32)
    rank = jnp.sum(gt + eqlt, axis=1, keepdims=True)
    dest_ref[0] = jnp.where(rank < NKP, b * NKP + rank, DUMP)
    scaled_ref[0] = xt_ref[0] * jnp.tanh(s_col)


_rank_call = pl.pallas_call(
    _rank_body,
    grid=(B, N // _NB_D),
    in_specs=[
        pl.BlockSpec((1, _NB_D, 1), lambda b, i: (b, i, 0)),
        pl.BlockSpec((1, 1, N), lambda b, i: (b, 0, 0)),
        pl.BlockSpec((1, _NB_D, C), lambda b, i: (b, i, 0)),
    ],
    out_specs=[
        pl.BlockSpec((1, _NB_D, 1), lambda b, i: (b, i, 0)),
        pl.BlockSpec((1, _NB_D, C), lambda b, i: (b, i, 0)),
    ],
    out_shape=[
        jax.ShapeDtypeStruct((B, N, 1), jnp.int32),
        jax.ShapeDtypeStruct((B, N, C), jnp.float32),
    ],
)


_info = plsc.get_sparse_core_info()
_NW = _info.num_cores * _info.num_subcores
_SLAB = 128
_mesh = plsc.VectorSubcoreMesh(core_axis_name="c", subcore_axis_name="s")

_G_ROWS = K * B * N
_G_PER_W = _G_ROWS // _NW
_G_SLABS = _G_PER_W // _SLAB


@functools.partial(
    pl.kernel,
    mesh=_mesh,
    out_type=jax.ShapeDtypeStruct((_G_ROWS, C), jnp.float32),
    scratch_types=[
        pltpu.VMEM((_SLAB,), jnp.int32),
        pltpu.VMEM((_SLAB, C), jnp.float32),
        pltpu.SemaphoreType.DMA,
    ],
)
def _sc_gather(table_hbm, idx_hbm, out_hbm, idx_v, rows_v, sem):
    wid = lax.axis_index("s") * _info.num_cores + lax.axis_index("c")
    wbase = wid * _G_PER_W

    def body(i, carry):
        base = wbase + i * _SLAB
        pltpu.sync_copy(idx_hbm.at[pl.ds(base, _SLAB)], idx_v)
        pltpu.async_copy(table_hbm.at[idx_v], rows_v, sem).wait()
        pltpu.sync_copy(rows_v, out_hbm.at[pl.ds(base, _SLAB)])
        return carry

    lax.fori_loop(0, _G_SLABS, body, 0)


_S_ROWS = B * N
_S_PER_W = _S_ROWS // _NW
_S_SLABS = _S_PER_W // _SLAB


@functools.partial(
    pl.kernel,
    mesh=_mesh,
    out_type=jax.ShapeDtypeStruct((DUMP + 8, C), jnp.float32),
    scratch_types=[
        pltpu.VMEM((_SLAB,), jnp.int32),
        pltpu.VMEM((_SLAB, C), jnp.float32),
        pltpu.SemaphoreType.DMA,
    ],
)
def _sc_scatter(rows_hbm, idx_hbm, out_hbm, idx_v, rows_v, sem):
    wid = lax.axis_index("s") * _info.num_cores + lax.axis_index("c")
    wbase = wid * _S_PER_W

    def body(i, carry):
        base = wbase + i * _SLAB
        pltpu.sync_copy(idx_hbm.at[pl.ds(base, _SLAB)], idx_v)
        pltpu.sync_copy(rows_hbm.at[pl.ds(base, _SLAB)], rows_v)
        pltpu.async_copy(rows_v, out_hbm.at[idx_v], sem).wait()
        return carry

    lax.fori_loop(0, _S_SLABS, body, 0)


def kernel(feat, W, b):
    feat_t = jnp.transpose(feat, (0, 2, 1))
    knn_idx = _knn_call(feat_t, feat)

    idx_t = jnp.transpose(knn_idx, (2, 0, 1)).reshape(_G_ROWS)
    nbr_flat = _sc_gather(feat_t.reshape(B * N, C), idx_t)
    nbr = nbr_flat.reshape(K, B, N, C)

    w_col = jnp.transpose(W)
    b_arr = b.reshape(1, 1)
    scores_col = _score_call(nbr, feat_t, w_col, b_arr)
    scores_row = jnp.transpose(scores_col, (0, 2, 1))

    dest, scaled = _rank_call(scores_col, scores_row, feat_t)
    out_buf = _sc_scatter(scaled.reshape(B * N, C), dest.reshape(B * N))
    return out_buf[:B * NKP].reshape(B, NKP, C)


# speedup vs baseline: 4.4216x; 1.0113x over previous
import functools

import jax
import jax.numpy as jnp
from jax import lax
from jax.experimental import pallas as pl
from jax.experimental.pallas import tpu as pltpu
from jax.experimental.pallas import tpu_sc as plsc

B, C, N, K = 4, 128, 2048, 16
NKP = 1024
DUMP = B * NKP

_PREC = "default"


_NB_A = 256


def _knn_body(xt_ref, x_ref, out_ref):
    b = pl.program_id(0)
    xtb = xt_ref[0]
    xb = x_ref[0]
    inner = -2.0 * jnp.dot(xtb, xb, precision=_PREC,
                           preferred_element_type=jnp.float32)
    xx_row = jnp.sum(xb * xb, axis=0, keepdims=True)
    xx_col = jnp.sum(xtb * xtb, axis=1, keepdims=True)
    dwork = -xx_col - inner - xx_row

    iota = lax.broadcasted_iota(jnp.int32, (_NB_A, N), 1)
    neg_inf = jnp.float32(-jnp.inf)
    bigi = jnp.int32(1 << 30)
    cols = []
    for t in range(K):
        rowmax = jnp.max(dwork, axis=1, keepdims=True)
        cand = jnp.where(dwork == rowmax, iota, bigi)
        mstar = jnp.min(cand, axis=1, keepdims=True)
        cols.append(mstar)
        if t < K - 1:
            dwork = jnp.where(iota == mstar, neg_inf, dwork)
    out_ref[0] = jnp.concatenate(cols, axis=1) + b * N


_knn_call = pl.pallas_call(
    _knn_body,
    grid=(B, N // _NB_A),
    in_specs=[
        pl.BlockSpec((1, _NB_A, C), lambda b, i: (b, i, 0)),
        pl.BlockSpec((1, C, N), lambda b, i: (b, 0, 0)),
    ],
    out_specs=pl.BlockSpec((1, _NB_A, K), lambda b, i: (b, i, 0)),
    out_shape=jax.ShapeDtypeStruct((B, N, K), jnp.int32),
)


_NB_C = 512


def _score_body(nbr_ref, xt_ref, w_ref, b_ref, out_ref):
    j = pl.program_id(2)
    nbrb = nbr_ref[0, 0]
    xtb = xt_ref[0]
    gf = jnp.concatenate([nbrb - xtb, xtb], axis=1)
    sc = jnp.dot(gf, w_ref[...], precision=_PREC,
                 preferred_element_type=jnp.float32) + b_ref[0, 0]

    @pl.when(j == 0)
    def _():
        out_ref[0] = sc

    @pl.when(j > 0)
    def _():
        out_ref[0] = jnp.maximum(out_ref[0], sc)


_score_call = pl.pallas_call(
    _score_body,
    grid=(B, N // _NB_C, K),
    in_specs=[
        pl.BlockSpec((1, 1, _NB_C, C), lambda b, i, j: (j, b, i, 0)),
        pl.BlockSpec((1, _NB_C, C), lambda b, i, j: (b, i, 0)),
        pl.BlockSpec((2 * C, 1), lambda b, i, j: (0, 0)),
        pl.BlockSpec((1, 1), lambda b, i, j: (0, 0)),
    ],
    out_specs=pl.BlockSpec((1, _NB_C, 1), lambda b, i, j: (b, i, 0)),
    out_shape=jax.ShapeDtypeStruct((B, N, 1), jnp.float32),
)


_NB_D = 512


def _rank_body(sc_ref, sr_ref, xt_ref, dest_ref, scaled_ref):
    b = pl.program_id(0)
    i = pl.program_id(1)
    s_col = jnp.maximum(sc_ref[0], 0.0)
    s_row = jnp.maximum(sr_ref[0], 0.0)
    gt = (s_row > s_col).astype(jnp.int32)
    ncol = i * _NB_D + lax.broadcasted_iota(jnp.int32, (_NB_D, 1), 0)
    mrow = lax.broadcasted_iota(jnp.int32, (_NB_D, N), 1)
    eqlt = ((s_row == s_col) & (mrow < ncol)).astype(jnp.int32)
    rank = jnp.sum(gt + eqlt, axis=1, keepdims=True)
    dest_ref[0] = jnp.where(rank < NKP, b * NKP + rank, DUMP)
    scaled_ref[0] = xt_ref[0] * jnp.tanh(s_col)


_rank_call = pl.pallas_call(
    _rank_body,
    grid=(B, N // _NB_D),
    in_specs=[
        pl.BlockSpec((1, _NB_D, 1), lambda b, i: (b, i, 0)),
        pl.BlockSpec((1, 1, N), lambda b, i: (b, 0, 0)),
        pl.BlockSpec((1, _NB_D, C), lambda b, i: (b, i, 0)),
    ],
    out_specs=[
        pl.BlockSpec((1, _NB_D, 1), lambda b, i: (b, i, 0)),
        pl.BlockSpec((1, _NB_D, C), lambda b, i: (b, i, 0)),
    ],
    out_shape=[
        jax.ShapeDtypeStruct((B, N, 1), jnp.int32),
        jax.ShapeDtypeStruct((B, N, C), jnp.float32),
    ],
)


_info = plsc.get_sparse_core_info()
_NW = _info.num_cores * _info.num_subcores
_mesh = plsc.VectorSubcoreMesh(core_axis_name="c", subcore_axis_name="s")

_G_ROWS = K * B * N
_G_PER_W = _G_ROWS // _NW
_TR = 128
_NT = _G_PER_W // _TR


@functools.partial(
    pl.kernel,
    mesh=_mesh,
    out_type=jax.ShapeDtypeStruct((_G_ROWS, C), jnp.float32),
    scratch_types=[
        pltpu.VMEM((_G_PER_W // 128, 128), jnp.int32),
        pltpu.VMEM((_TR, C), jnp.float32),
        pltpu.VMEM((_TR, C), jnp.float32),
        pltpu.SemaphoreType.DMA,
        pltpu.SemaphoreType.DMA,
        pltpu.SemaphoreType.DMA,
        pltpu.SemaphoreType.DMA,
    ],
)
def _sc_gather(table_hbm, idx_hbm, out_hbm, idx_all, b0, b1, gs0, gs1, os0, os1):
    wid = lax.axis_index("s") * _info.num_cores + lax.axis_index("c")
    wbase = wid * _G_PER_W
    pltpu.sync_copy(idx_hbm.at[pl.ds(wid * (_G_PER_W // 128), _G_PER_W // 128)],
                    idx_all)

    def gstart(t, buf, sem):
        pltpu.async_copy(table_hbm.at[idx_all.at[t]], buf, sem)

    def gwait(buf, sem):
        pltpu.make_async_copy(out_hbm.at[pl.ds(0, _TR)], buf, sem).wait()

    def sstart(t, buf, sem):
        pltpu.async_copy(buf, out_hbm.at[pl.ds(wbase + t * _TR, _TR)], sem)

    def swait(buf, sem):
        pltpu.make_async_copy(buf, out_hbm.at[pl.ds(0, _TR)], sem).wait()

    gstart(0, b0, gs0)

    def outer(o, carry):
        i = 2 * o
        gwait(b0, gs0)

        @pl.when(o > 0)
        def _():
            swait(b1, os1)

        gstart(i + 1, b1, gs1)
        sstart(i, b0, os0)
        gwait(b1, gs1)

        @pl.when(o < _NT // 2 - 1)
        def _():
            swait(b0, os0)
            gstart(i + 2, b0, gs0)

        sstart(i + 1, b1, os1)
        return carry

    lax.fori_loop(0, _NT // 2, outer, 0)
    swait(b0, os0)
    swait(b1, os1)


_S_ROWS = B * N
_S_PER_W = _S_ROWS // _NW


@functools.partial(
    pl.kernel,
    mesh=_mesh,
    out_type=jax.ShapeDtypeStruct((DUMP + 8, C), jnp.float32),
    scratch_types=[
        pltpu.VMEM((128,), jnp.int32),
        pltpu.VMEM((128,), jnp.int32),
        pltpu.VMEM((_S_PER_W, C), jnp.float32),
        pltpu.SemaphoreType.DMA,
    ],
)
def _sc_scatter(rows_hbm, idx_hbm, out_hbm, idx_v0, idx_v1, rows_v, sem):
    wid = lax.axis_index("s") * _info.num_cores + lax.axis_index("c")
    wbase = wid * _S_PER_W
    pltpu.sync_copy(idx_hbm.at[pl.ds(wbase, 128)], idx_v0)
    pltpu.sync_copy(idx_hbm.at[pl.ds(wbase + 128, 128)], idx_v1)
    pltpu.sync_copy(rows_hbm.at[pl.ds(wbase, _S_PER_W)], rows_v)
    pltpu.async_copy(rows_v.at[pl.ds(0, 128)], out_hbm.at[idx_v0], sem)
    pltpu.async_copy(rows_v.at[pl.ds(128, 128)], out_hbm.at[idx_v1], sem)
    pltpu.make_async_copy(rows_v, out_hbm.at[pl.ds(0, _S_PER_W)], sem).wait()


def kernel(feat, W, b):
    feat_t = jnp.transpose(feat, (0, 2, 1))
    knn_idx = _knn_call(feat_t, feat)

    idx_t = jnp.transpose(knn_idx, (2, 0, 1)).reshape(_G_ROWS // 128, 128)
    nbr_flat = _sc_gather(feat_t.reshape(B * N, C), idx_t)
    nbr = nbr_flat.reshape(K, B, N, C)

    w_col = jnp.transpose(W)
    b_arr = b.reshape(1, 1)
    scores_col = _score_call(nbr, feat_t, w_col, b_arr)
    scores_row = jnp.transpose(scores_col, (0, 2, 1))

    dest, scaled = _rank_call(scores_col, scores_row, feat_t)
    out_buf = _sc_scatter(scaled.reshape(B * N, C), dest.reshape(B * N))
    return out_buf[:B * NKP].reshape(B, NKP, C)


# speedup vs baseline: 6.0042x; 1.3579x over previous
import functools

import jax
import jax.numpy as jnp
from jax import lax
from jax.experimental import pallas as pl
from jax.experimental.pallas import tpu as pltpu
from jax.experimental.pallas import tpu_sc as plsc

B, C, N, K = 4, 128, 2048, 16
NKP = 1024
DUMP = B * NKP

_PREC = "default"


_NB_A = 256


def _knn_body(xt_ref, x_ref, w_ref, out_ref):
    b = pl.program_id(0)
    xtb = xt_ref[0]
    xb = x_ref[0]
    inner = -2.0 * jnp.dot(xtb, xb, precision=_PREC,
                           preferred_element_type=jnp.float32)
    xx_row = jnp.sum(xb * xb, axis=0, keepdims=True)
    xx_col = jnp.sum(xtb * xtb, axis=1, keepdims=True)
    dwork = -xx_col - inner - xx_row
    s_row = jnp.dot(w_ref[:, :C], xb, precision=_PREC,
                    preferred_element_type=jnp.float32)

    iota = lax.broadcasted_iota(jnp.int32, (_NB_A, N), 1)
    neg_inf = jnp.float32(-jnp.inf)
    bigi = jnp.int32(1 << 30)
    for t in range(K):
        rowmax = jnp.max(dwork, axis=1, keepdims=True)
        cand = jnp.where(dwork == rowmax, iota, bigi)
        mstar = jnp.min(cand, axis=1, keepdims=True)
        dwork = jnp.where(iota == mstar, neg_inf, dwork)
    sm = jnp.where(dwork == neg_inf, s_row, neg_inf)
    cols = []
    for _ in range(2):
        smax = jnp.max(sm, axis=1, keepdims=True)
        cand = jnp.where(sm == smax, iota, bigi)
        mstar = jnp.min(cand, axis=1, keepdims=True)
        cols.append(mstar)
        sm = jnp.where(iota == mstar, neg_inf, sm)
    out_ref[0] = jnp.concatenate(cols, axis=1) + b * N


_knn_call = pl.pallas_call(
    _knn_body,
    grid=(B, N // _NB_A),
    in_specs=[
        pl.BlockSpec((1, _NB_A, C), lambda b, i: (b, i, 0)),
        pl.BlockSpec((1, C, N), lambda b, i: (b, 0, 0)),
        pl.BlockSpec((1, 2 * C), lambda b, i: (0, 0)),
    ],
    out_specs=pl.BlockSpec((1, _NB_A, 2), lambda b, i: (b, i, 0)),
    out_shape=jax.ShapeDtypeStruct((B, N, 2), jnp.int32),
)


_NB_C = 512


def _score_body(nbr_ref, xt_ref, w_ref, b_ref, out_ref):
    xtb = xt_ref[0]
    bias = b_ref[0, 0]
    gf0 = jnp.concatenate([nbr_ref[0, 0] - xtb, xtb], axis=1)
    sc0 = jnp.dot(gf0, w_ref[...], precision=_PREC,
                  preferred_element_type=jnp.float32) + bias
    gf1 = jnp.concatenate([nbr_ref[1, 0] - xtb, xtb], axis=1)
    sc1 = jnp.dot(gf1, w_ref[...], precision=_PREC,
                  preferred_element_type=jnp.float32) + bias
    out_ref[0] = jnp.maximum(sc0, sc1)


_score_call = pl.pallas_call(
    _score_body,
    grid=(B, N // _NB_C),
    in_specs=[
        pl.BlockSpec((2, 1, _NB_C, C), lambda b, i: (0, b, i, 0)),
        pl.BlockSpec((1, _NB_C, C), lambda b, i: (b, i, 0)),
        pl.BlockSpec((2 * C, 1), lambda b, i: (0, 0)),
        pl.BlockSpec((1, 1), lambda b, i: (0, 0)),
    ],
    out_specs=pl.BlockSpec((1, _NB_C, 1), lambda b, i: (b, i, 0)),
    out_shape=jax.ShapeDtypeStruct((B, N, 1), jnp.float32),
)


_NB_D = 512


def _rank_body(sc_ref, sr_ref, xt_ref, dest_ref, scaled_ref):
    b = pl.program_id(0)
    i = pl.program_id(1)
    s_col = jnp.maximum(sc_ref[0], 0.0)
    s_row = jnp.maximum(sr_ref[0], 0.0)
    gt = (s_row > s_col).astype(jnp.int32)
    ncol = i * _NB_D + lax.broadcasted_iota(jnp.int32, (_NB_D, 1), 0)
    mrow = lax.broadcasted_iota(jnp.int32, (_NB_D, N), 1)
    eqlt = ((s_row == s_col) & (mrow < ncol)).astype(jnp.int32)
    rank = jnp.sum(gt + eqlt, axis=1, keepdims=True)
    dest_ref[0] = jnp.where(rank < NKP, b * NKP + rank, DUMP)
    scaled_ref[0] = xt_ref[0] * jnp.tanh(s_col)


_rank_call = pl.pallas_call(
    _rank_body,
    grid=(B, N // _NB_D),
    in_specs=[
        pl.BlockSpec((1, _NB_D, 1), lambda b, i: (b, i, 0)),
        pl.BlockSpec((1, 1, N), lambda b, i: (b, 0, 0)),
        pl.BlockSpec((1, _NB_D, C), lambda b, i: (b, i, 0)),
    ],
    out_specs=[
        pl.BlockSpec((1, _NB_D, 1), lambda b, i: (b, i, 0)),
        pl.BlockSpec((1, _NB_D, C), lambda b, i: (b, i, 0)),
    ],
    out_shape=[
        jax.ShapeDtypeStruct((B, N, 1), jnp.int32),
        jax.ShapeDtypeStruct((B, N, C), jnp.float32),
    ],
)


_info = plsc.get_sparse_core_info()
_NW = _info.num_cores * _info.num_subcores
_mesh = plsc.VectorSubcoreMesh(core_axis_name="c", subcore_axis_name="s")

_G_ROWS = 2 * B * N
_G_PER_W = _G_ROWS // _NW
_TR = 128
_NT = _G_PER_W // _TR


@functools.partial(
    pl.kernel,
    mesh=_mesh,
    out_type=jax.ShapeDtypeStruct((_G_ROWS, C), jnp.float32),
    scratch_types=[
        pltpu.VMEM((_G_PER_W // 128, 128), jnp.int32),
        pltpu.VMEM((_TR, C), jnp.float32),
        pltpu.VMEM((_TR, C), jnp.float32),
        pltpu.SemaphoreType.DMA,
        pltpu.SemaphoreType.DMA,
        pltpu.SemaphoreType.DMA,
        pltpu.SemaphoreType.DMA,
    ],
)
def _sc_gather(table_hbm, idx_hbm, out_hbm, idx_all, b0, b1, gs0, gs1, os0, os1):
    wid = lax.axis_index("s") * _info.num_cores + lax.axis_index("c")
    wbase = wid * _G_PER_W
    pltpu.sync_copy(idx_hbm.at[pl.ds(wid * (_G_PER_W // 128), _G_PER_W // 128)],
                    idx_all)

    def gstart(t, buf, sem):
        pltpu.async_copy(table_hbm.at[idx_all.at[t]], buf, sem)

    def gwait(buf, sem):
        pltpu.make_async_copy(out_hbm.at[pl.ds(0, _TR)], buf, sem).wait()

    def sstart(t, buf, sem):
        pltpu.async_copy(buf, out_hbm.at[pl.ds(wbase + t * _TR, _TR)], sem)

    def swait(buf, sem):
        pltpu.make_async_copy(buf, out_hbm.at[pl.ds(0, _TR)], sem).wait()

    gstart(0, b0, gs0)

    def outer(o, carry):
        i = 2 * o
        gwait(b0, gs0)

        @pl.when(o > 0)
        def _():
            swait(b1, os1)

        gstart(i + 1, b1, gs1)
        sstart(i, b0, os0)
        gwait(b1, gs1)

        @pl.when(o < _NT // 2 - 1)
        def _():
            swait(b0, os0)
            gstart(i + 2, b0, gs0)

        sstart(i + 1, b1, os1)
        return carry

    lax.fori_loop(0, _NT // 2, outer, 0)
    swait(b0, os0)
    swait(b1, os1)


_S_ROWS = B * N
_S_PER_W = _S_ROWS // _NW


@functools.partial(
    pl.kernel,
    mesh=_mesh,
    out_type=jax.ShapeDtypeStruct((DUMP + 8, C), jnp.float32),
    scratch_types=[
        pltpu.VMEM((128,), jnp.int32),
        pltpu.VMEM((128,), jnp.int32),
        pltpu.VMEM((_S_PER_W, C), jnp.float32),
        pltpu.SemaphoreType.DMA,
    ],
)
def _sc_scatter(rows_hbm, idx_hbm, out_hbm, idx_v0, idx_v1, rows_v, sem):
    wid = lax.axis_index("s") * _info.num_cores + lax.axis_index("c")
    wbase = wid * _S_PER_W
    pltpu.sync_copy(idx_hbm.at[pl.ds(wbase, 128)], idx_v0)
    pltpu.sync_copy(idx_hbm.at[pl.ds(wbase + 128, 128)], idx_v1)
    pltpu.sync_copy(rows_hbm.at[pl.ds(wbase, _S_PER_W)], rows_v)
    pltpu.async_copy(rows_v.at[pl.ds(0, 128)], out_hbm.at[idx_v0], sem)
    pltpu.async_copy(rows_v.at[pl.ds(128, 128)], out_hbm.at[idx_v1], sem)
    pltpu.make_async_copy(rows_v, out_hbm.at[pl.ds(0, _S_PER_W)], sem).wait()


def kernel(feat, W, b):
    feat_t = jnp.transpose(feat, (0, 2, 1))
    knn_idx = _knn_call(feat_t, feat, W)

    idx_t = jnp.transpose(knn_idx, (2, 0, 1)).reshape(_G_ROWS // 128, 128)
    nbr_flat = _sc_gather(feat_t.reshape(B * N, C), idx_t)
    nbr = nbr_flat.reshape(2, B, N, C)

    w_col = jnp.transpose(W)
    b_arr = b.reshape(1, 1)
    scores_col = _score_call(nbr, feat_t, w_col, b_arr)
    scores_row = jnp.transpose(scores_col, (0, 2, 1))

    dest, scaled = _rank_call(scores_col, scores_row, feat_t)
    out_buf = _sc_scatter(scaled.reshape(B * N, C), dest.reshape(B * N))
    return out_buf[:B * NKP].reshape(B, NKP, C)


# speedup vs baseline: 8.1182x; 1.3521x over previous
import functools

import jax
import jax.numpy as jnp
from jax import lax
from jax.experimental import pallas as pl
from jax.experimental.pallas import tpu as pltpu
from jax.experimental.pallas import tpu_sc as plsc

B, C, N, K = 4, 128, 2048, 16
NKP = 1024
DUMP = B * NKP

_PREC = "default"


_NB_A = 256


def _knn_body(xt_ref, x_ref, w_ref, out_ref):
    b = pl.program_id(0)
    xtb = xt_ref[0]
    xb = x_ref[0]
    inner = -2.0 * jnp.dot(xtb, xb, precision=_PREC,
                           preferred_element_type=jnp.float32)
    xx_row = jnp.sum(xb * xb, axis=0, keepdims=True)
    xx_col = jnp.sum(xtb * xtb, axis=1, keepdims=True)
    dwork = -xx_col - inner - xx_row
    s_row = jnp.dot(w_ref[:, :C], xb, precision=_PREC,
                    preferred_element_type=jnp.float32)

    iota = lax.broadcasted_iota(jnp.int32, (_NB_A, N), 1)
    neg_inf = jnp.float32(-jnp.inf)
    bigi = jnp.int32(1 << 30)
    for t in range(K):
        rowmax = jnp.max(dwork, axis=1, keepdims=True)
        cand = jnp.where(dwork == rowmax, iota, bigi)
        mstar = jnp.min(cand, axis=1, keepdims=True)
        dwork = jnp.where(iota == mstar, neg_inf, dwork)
    sm = jnp.where(dwork == neg_inf, s_row, neg_inf)
    cols = []
    for _ in range(2):
        smax = jnp.max(sm, axis=1, keepdims=True)
        cand = jnp.where(sm == smax, iota, bigi)
        mstar = jnp.min(cand, axis=1, keepdims=True)
        cols.append(mstar)
        sm = jnp.where(iota == mstar, neg_inf, sm)
    out_ref[0] = jnp.concatenate(cols, axis=1) + b * N


_knn_call = pl.pallas_call(
    _knn_body,
    grid=(B, N // _NB_A),
    in_specs=[
        pl.BlockSpec((1, _NB_A, C), lambda b, i: (b, i, 0)),
        pl.BlockSpec((1, C, N), lambda b, i: (b, 0, 0)),
        pl.BlockSpec((1, 2 * C), lambda b, i: (0, 0)),
    ],
    out_specs=pl.BlockSpec((1, _NB_A, 2), lambda b, i: (b, i, 0)),
    out_shape=jax.ShapeDtypeStruct((B, N, 2), jnp.int32),
)


_NB_C = 512


def _score_body(nbr_ref, xt_ref, w_ref, b_ref, out_ref):
    xtb = xt_ref[0]
    bias = b_ref[0, 0]
    gf0 = jnp.concatenate([nbr_ref[0, 0] - xtb, xtb], axis=1)
    sc0 = jnp.dot(gf0, w_ref[...], precision=_PREC,
                  preferred_element_type=jnp.float32) + bias
    gf1 = jnp.concatenate([nbr_ref[1, 0] - xtb, xtb], axis=1)
    sc1 = jnp.dot(gf1, w_ref[...], precision=_PREC,
                  preferred_element_type=jnp.float32) + bias
    out_ref[0] = jnp.maximum(sc0, sc1)


_score_call = pl.pallas_call(
    _score_body,
    grid=(B, N // _NB_C),
    in_specs=[
        pl.BlockSpec((2, 1, _NB_C, C), lambda b, i: (0, b, i, 0)),
        pl.BlockSpec((1, _NB_C, C), lambda b, i: (b, i, 0)),
        pl.BlockSpec((2 * C, 1), lambda b, i: (0, 0)),
        pl.BlockSpec((1, 1), lambda b, i: (0, 0)),
    ],
    out_specs=pl.BlockSpec((1, _NB_C, 1), lambda b, i: (b, i, 0)),
    out_shape=jax.ShapeDtypeStruct((B, N, 1), jnp.float32),
)


_NB_D = 512


def _rank_body(sc_ref, sr_ref, xt_ref, dest_ref, scaled_ref):
    b = pl.program_id(0)
    i = pl.program_id(1)
    s_col = jnp.maximum(sc_ref[0], 0.0)
    s_row = jnp.maximum(sr_ref[0], 0.0)
    gt = (s_row > s_col).astype(jnp.int32)
    ncol = i * _NB_D + lax.broadcasted_iota(jnp.int32, (_NB_D, 1), 0)
    mrow = lax.broadcasted_iota(jnp.int32, (_NB_D, N), 1)
    eqlt = ((s_row == s_col) & (mrow < ncol)).astype(jnp.int32)
    rank = jnp.sum(gt + eqlt, axis=1, keepdims=True)
    flat_n = b * N + ncol
    dest_ref[0] = jnp.where(rank < NKP, b * NKP + rank, DUMP + flat_n)
    scaled_ref[0] = xt_ref[0] * jnp.tanh(s_col)


_rank_call = pl.pallas_call(
    _rank_body,
    grid=(B, N // _NB_D),
    in_specs=[
        pl.BlockSpec((1, _NB_D, 1), lambda b, i: (b, i, 0)),
        pl.BlockSpec((1, 1, N), lambda b, i: (b, 0, 0)),
        pl.BlockSpec((1, _NB_D, C), lambda b, i: (b, i, 0)),
    ],
    out_specs=[
        pl.BlockSpec((1, _NB_D, 1), lambda b, i: (b, i, 0)),
        pl.BlockSpec((1, _NB_D, C), lambda b, i: (b, i, 0)),
    ],
    out_shape=[
        jax.ShapeDtypeStruct((B, N, 1), jnp.int32),
        jax.ShapeDtypeStruct((B, N, C), jnp.float32),
    ],
)


_info = plsc.get_sparse_core_info()
_NW = _info.num_cores * _info.num_subcores
_mesh = plsc.VectorSubcoreMesh(core_axis_name="c", subcore_axis_name="s")

_G_ROWS = 2 * B * N
_G_PER_W = _G_ROWS // _NW
_TR = 128
_NT = _G_PER_W // _TR


@functools.partial(
    pl.kernel,
    mesh=_mesh,
    out_type=jax.ShapeDtypeStruct((_G_ROWS, C), jnp.float32),
    scratch_types=[
        pltpu.VMEM((_G_PER_W // 128, 128), jnp.int32),
        pltpu.VMEM((_TR, C), jnp.float32),
        pltpu.VMEM((_TR, C), jnp.float32),
        pltpu.SemaphoreType.DMA,
        pltpu.SemaphoreType.DMA,
        pltpu.SemaphoreType.DMA,
        pltpu.SemaphoreType.DMA,
    ],
)
def _sc_gather(table_hbm, idx_hbm, out_hbm, idx_all, b0, b1, gs0, gs1, os0, os1):
    wid = lax.axis_index("s") * _info.num_cores + lax.axis_index("c")
    wbase = wid * _G_PER_W
    pltpu.sync_copy(idx_hbm.at[pl.ds(wid * (_G_PER_W // 128), _G_PER_W // 128)],
                    idx_all)

    def gstart(t, buf, sem):
        pltpu.async_copy(table_hbm.at[idx_all.at[t]], buf, sem)

    def gwait(buf, sem):
        pltpu.make_async_copy(out_hbm.at[pl.ds(0, _TR)], buf, sem).wait()

    def sstart(t, buf, sem):
        pltpu.async_copy(buf, out_hbm.at[pl.ds(wbase + t * _TR, _TR)], sem)

    def swait(buf, sem):
        pltpu.make_async_copy(buf, out_hbm.at[pl.ds(0, _TR)], sem).wait()

    gstart(0, b0, gs0)

    def outer(o, carry):
        i = 2 * o
        gwait(b0, gs0)

        @pl.when(o > 0)
        def _():
            swait(b1, os1)

        gstart(i + 1, b1, gs1)
        sstart(i, b0, os0)
        gwait(b1, gs1)

        @pl.when(o < _NT // 2 - 1)
        def _():
            swait(b0, os0)
            gstart(i + 2, b0, gs0)

        sstart(i + 1, b1, os1)
        return carry

    lax.fori_loop(0, _NT // 2, outer, 0)
    swait(b0, os0)
    swait(b1, os1)


_S_ROWS = B * N
_S_PER_W = _S_ROWS // _NW


@functools.partial(
    pl.kernel,
    mesh=_mesh,
    out_type=jax.ShapeDtypeStruct((DUMP + B * N, C), jnp.float32),
    scratch_types=[
        pltpu.VMEM((128,), jnp.int32),
        pltpu.VMEM((128,), jnp.int32),
        pltpu.VMEM((_S_PER_W, C), jnp.float32),
        pltpu.SemaphoreType.DMA,
    ],
)
def _sc_scatter(rows_hbm, idx_hbm, out_hbm, idx_v0, idx_v1, rows_v, sem):
    wid = lax.axis_index("s") * _info.num_cores + lax.axis_index("c")
    wbase = wid * _S_PER_W
    pltpu.sync_copy(idx_hbm.at[pl.ds(wbase, 128)], idx_v0)
    pltpu.sync_copy(idx_hbm.at[pl.ds(wbase + 128, 128)], idx_v1)
    pltpu.sync_copy(rows_hbm.at[pl.ds(wbase, _S_PER_W)], rows_v)
    pltpu.async_copy(rows_v.at[pl.ds(0, 128)], out_hbm.at[idx_v0], sem)
    pltpu.async_copy(rows_v.at[pl.ds(128, 128)], out_hbm.at[idx_v1], sem)
    pltpu.make_async_copy(rows_v, out_hbm.at[pl.ds(0, _S_PER_W)], sem).wait()


def kernel(feat, W, b):
    feat_t = jnp.transpose(feat, (0, 2, 1))
    knn_idx = _knn_call(feat_t, feat, W)

    idx_t = jnp.transpose(knn_idx, (2, 0, 1)).reshape(_G_ROWS // 128, 128)
    nbr_flat = _sc_gather(feat_t.reshape(B * N, C), idx_t)
    nbr = nbr_flat.reshape(2, B, N, C)

    w_col = jnp.transpose(W)
    b_arr = b.reshape(1, 1)
    scores_col = _score_call(nbr, feat_t, w_col, b_arr)
    scores_row = jnp.transpose(scores_col, (0, 2, 1))

    dest, scaled = _rank_call(scores_col, scores_row, feat_t)
    out_buf = _sc_scatter(scaled.reshape(B * N, C), dest.reshape(B * N))
    return out_buf[:B * NKP].reshape(B, NKP, C)


# speedup vs baseline: 12.7533x; 1.5709x over previous
import functools

import jax
import jax.numpy as jnp
from jax import lax
from jax.experimental import pallas as pl
from jax.experimental.pallas import tpu as pltpu
from jax.experimental.pallas import tpu_sc as plsc

B, C, N, K = 4, 128, 2048, 16
NKP = 1024
DUMP = B * NKP

_PREC = "default"


_NB_A = 256


def _knn_body(xt_ref, x_ref, w_ref, out_ref):
    b = pl.program_id(0)
    xtb = xt_ref[0]
    xb = x_ref[0]
    inner = -2.0 * jnp.dot(xtb, xb, precision=_PREC,
                           preferred_element_type=jnp.float32)
    xx_row = jnp.sum(xb * xb, axis=0, keepdims=True)
    xx_col = jnp.sum(xtb * xtb, axis=1, keepdims=True)
    dwork = -xx_col - inner - xx_row
    s_row = jnp.dot(w_ref[:, :C], xb, precision=_PREC,
                    preferred_element_type=jnp.float32)

    iota = lax.broadcasted_iota(jnp.int32, (_NB_A, N), 1)
    neg_inf = jnp.float32(-jnp.inf)
    bigi = jnp.int32(1 << 30)

    def _top2_by_s(mask):
        sm = jnp.where(mask, s_row, neg_inf)
        cols = []
        for _ in range(2):
            smax = jnp.max(sm, axis=1, keepdims=True)
            cand = jnp.where(sm == smax, iota, bigi)
            mstar = jnp.min(cand, axis=1, keepdims=True)
            cols.append(mstar)
            sm = jnp.where(iota == mstar, neg_inf, sm)
        return jnp.concatenate(cols, axis=1) + b * N

    dfast = dwork
    for _ in range(K):
        rowmax = jnp.max(dfast, axis=1, keepdims=True)
        dfast = jnp.where(dfast == rowmax, neg_inf, dfast)
    cnt = jnp.sum((dfast == neg_inf).astype(jnp.int32), axis=1)
    exact = jnp.max(cnt) == K

    @pl.when(exact)
    def _():
        out_ref[0] = _top2_by_s(dfast == neg_inf)

    @pl.when(jnp.logical_not(exact))
    def _():
        dslow = dwork
        for t in range(K):
            rowmax = jnp.max(dslow, axis=1, keepdims=True)
            cand = jnp.where(dslow == rowmax, iota, bigi)
            mstar = jnp.min(cand, axis=1, keepdims=True)
            dslow = jnp.where(iota == mstar, neg_inf, dslow)
        out_ref[0] = _top2_by_s(dslow == neg_inf)


_knn_call = pl.pallas_call(
    _knn_body,
    grid=(B, N // _NB_A),
    in_specs=[
        pl.BlockSpec((1, _NB_A, C), lambda b, i: (b, i, 0)),
        pl.BlockSpec((1, C, N), lambda b, i: (b, 0, 0)),
        pl.BlockSpec((1, 2 * C), lambda b, i: (0, 0)),
    ],
    out_specs=pl.BlockSpec((1, _NB_A, 2), lambda b, i: (b, i, 0)),
    out_shape=jax.ShapeDtypeStruct((B, N, 2), jnp.int32),
)


_NB_C = 512


def _score_body(nbr_ref, xt_ref, w_ref, b_ref, out_ref):
    xtb = xt_ref[0]
    bias = b_ref[0, 0]
    gf0 = jnp.concatenate([nbr_ref[0, 0] - xtb, xtb], axis=1)
    sc0 = jnp.dot(gf0, w_ref[...], precision=_PREC,
                  preferred_element_type=jnp.float32) + bias
    gf1 = jnp.concatenate([nbr_ref[1, 0] - xtb, xtb], axis=1)
    sc1 = jnp.dot(gf1, w_ref[...], precision=_PREC,
                  preferred_element_type=jnp.float32) + bias
    out_ref[0] = jnp.maximum(sc0, sc1)


_score_call = pl.pallas_call(
    _score_body,
    grid=(B, N // _NB_C),
    in_specs=[
        pl.BlockSpec((2, 1, _NB_C, C), lambda b, i: (0, b, i, 0)),
        pl.BlockSpec((1, _NB_C, C), lambda b, i: (b, i, 0)),
        pl.BlockSpec((2 * C, 1), lambda b, i: (0, 0)),
        pl.BlockSpec((1, 1), lambda b, i: (0, 0)),
    ],
    out_specs=pl.BlockSpec((1, _NB_C, 1), lambda b, i: (b, i, 0)),
    out_shape=jax.ShapeDtypeStruct((B, N, 1), jnp.float32),
)


_NB_D = 512


def _rank_body(sc_ref, sr_ref, xt_ref, dest_ref, scaled_ref):
    b = pl.program_id(0)
    i = pl.program_id(1)
    s_col = jnp.maximum(sc_ref[0], 0.0)
    s_row = jnp.maximum(sr_ref[0], 0.0)
    gt = (s_row > s_col).astype(jnp.int32)
    ncol = i * _NB_D + lax.broadcasted_iota(jnp.int32, (_NB_D, 1), 0)
    mrow = lax.broadcasted_iota(jnp.int32, (_NB_D, N), 1)
    eqlt = ((s_row == s_col) & (mrow < ncol)).astype(jnp.int32)
    rank = jnp.sum(gt + eqlt, axis=1, keepdims=True)
    flat_n = b * N + ncol
    dest_ref[0] = jnp.where(rank < NKP, b * NKP + rank, DUMP + flat_n)
    scaled_ref[0] = xt_ref[0] * jnp.tanh(s_col)


_rank_call = pl.pallas_call(
    _rank_body,
    grid=(B, N // _NB_D),
    in_specs=[
        pl.BlockSpec((1, _NB_D, 1), lambda b, i: (b, i, 0)),
        pl.BlockSpec((1, 1, N), lambda b, i: (b, 0, 0)),
        pl.BlockSpec((1, _NB_D, C), lambda b, i: (b, i, 0)),
    ],
    out_specs=[
        pl.BlockSpec((1, _NB_D, 1), lambda b, i: (b, i, 0)),
        pl.BlockSpec((1, _NB_D, C), lambda b, i: (b, i, 0)),
    ],
    out_shape=[
        jax.ShapeDtypeStruct((B, N, 1), jnp.int32),
        jax.ShapeDtypeStruct((B, N, C), jnp.float32),
    ],
)


_info = plsc.get_sparse_core_info()
_NW = _info.num_cores * _info.num_subcores
_mesh = plsc.VectorSubcoreMesh(core_axis_name="c", subcore_axis_name="s")

_G_ROWS = 2 * B * N
_G_PER_W = _G_ROWS // _NW
_TR = 128
_NT = _G_PER_W // _TR


@functools.partial(
    pl.kernel,
    mesh=_mesh,
    out_type=jax.ShapeDtypeStruct((_G_ROWS, C), jnp.float32),
    scratch_types=[
        pltpu.VMEM((_G_PER_W // 128, 128), jnp.int32),
        pltpu.VMEM((_TR, C), jnp.float32),
        pltpu.VMEM((_TR, C), jnp.float32),
        pltpu.SemaphoreType.DMA,
        pltpu.SemaphoreType.DMA,
        pltpu.SemaphoreType.DMA,
        pltpu.SemaphoreType.DMA,
    ],
)
def _sc_gather(table_hbm, idx_hbm, out_hbm, idx_all, b0, b1, gs0, gs1, os0, os1):
    wid = lax.axis_index("s") * _info.num_cores + lax.axis_index("c")
    wbase = wid * _G_PER_W
    pltpu.sync_copy(idx_hbm.at[pl.ds(wid * (_G_PER_W // 128), _G_PER_W // 128)],
                    idx_all)

    def gstart(t, buf, sem):
        pltpu.async_copy(table_hbm.at[idx_all.at[t]], buf, sem)

    def gwait(buf, sem):
        pltpu.make_async_copy(out_hbm.at[pl.ds(0, _TR)], buf, sem).wait()

    def sstart(t, buf, sem):
        pltpu.async_copy(buf, out_hbm.at[pl.ds(wbase + t * _TR, _TR)], sem)

    def swait(buf, sem):
        pltpu.make_async_copy(buf, out_hbm.at[pl.ds(0, _TR)], sem).wait()

    gstart(0, b0, gs0)

    def outer(o, carry):
        i = 2 * o
        gwait(b0, gs0)

        @pl.when(o > 0)
        def _():
            swait(b1, os1)

        gstart(i + 1, b1, gs1)
        sstart(i, b0, os0)
        gwait(b1, gs1)

        @pl.when(o < _NT // 2 - 1)
        def _():
            swait(b0, os0)
            gstart(i + 2, b0, gs0)

        sstart(i + 1, b1, os1)
        return carry

    lax.fori_loop(0, _NT // 2, outer, 0)
    swait(b0, os0)
    swait(b1, os1)


_S_ROWS = B * N
_S_PER_W = _S_ROWS // _NW


@functools.partial(
    pl.kernel,
    mesh=_mesh,
    out_type=jax.ShapeDtypeStruct((DUMP + B * N, C), jnp.float32),
    scratch_types=[
        pltpu.VMEM((128,), jnp.int32),
        pltpu.VMEM((128,), jnp.int32),
        pltpu.VMEM((_S_PER_W, C), jnp.float32),
        pltpu.SemaphoreType.DMA,
    ],
)
def _sc_scatter(rows_hbm, idx_hbm, out_hbm, idx_v0, idx_v1, rows_v, sem):
    wid = lax.axis_index("s") * _info.num_cores + lax.axis_index("c")
    wbase = wid * _S_PER_W
    pltpu.sync_copy(idx_hbm.at[pl.ds(wbase, 128)], idx_v0)
    pltpu.sync_copy(idx_hbm.at[pl.ds(wbase + 128, 128)], idx_v1)
    pltpu.sync_copy(rows_hbm.at[pl.ds(wbase, _S_PER_W)], rows_v)
    pltpu.async_copy(rows_v.at[pl.ds(0, 128)], out_hbm.at[idx_v0], sem)
    pltpu.async_copy(rows_v.at[pl.ds(128, 128)], out_hbm.at[idx_v1], sem)
    pltpu.make_async_copy(rows_v, out_hbm.at[pl.ds(0, _S_PER_W)], sem).wait()


def kernel(feat, W, b):
    feat_t = jnp.transpose(feat, (0, 2, 1))
    knn_idx = _knn_call(feat_t, feat, W)

    idx_t = jnp.transpose(knn_idx, (2, 0, 1)).reshape(_G_ROWS // 128, 128)
    nbr_flat = _sc_gather(feat_t.reshape(B * N, C), idx_t)
    nbr = nbr_flat.reshape(2, B, N, C)

    w_col = jnp.transpose(W)
    b_arr = b.reshape(1, 1)
    scores_col = _score_call(nbr, feat_t, w_col, b_arr)
    scores_row = jnp.transpose(scores_col, (0, 2, 1))

    dest, scaled = _rank_call(scores_col, scores_row, feat_t)
    out_buf = _sc_scatter(scaled.reshape(B * N, C), dest.reshape(B * N))
    return out_buf[:B * NKP].reshape(B, NKP, C)
